# Initial kernel scaffold; baseline (speedup 1.0000x reference)
import jax
import jax.numpy as jnp
from jax.experimental import pallas as pl


def kernel(x, edge_index, edge_attr, batch, W1, b1, Wk1, bk1, Wk2, bk2, Wk3, bk3, root, cbias, W2, b2):
    raise NotImplementedError("write your pallas kernel here")


# speedup vs baseline: 3.5372x; 3.5372x over previous
import functools

import jax
import jax.numpy as jnp
from jax import lax
from jax.experimental import pallas as pl
from jax.experimental.pallas import tpu as pltpu
from jax.experimental.pallas import tpu_sc as plsc

N = 10000
E = 320000
G = 16
DIM_IN = 128
DN = 16

NW = 32
EPW = E // NW
CH = 2000
NCH = EPW // CH

NB_N = 10
BN = N // NB_N
BE = 2000
NBE = E // BE

NP = 10112


def _f32(x):
    return jnp.dot(x[0], x[1], preferred_element_type=jnp.float32)


def _h0_body(x_ref, w_ref, b_ref, o_ref):
    o_ref[...] = jnp.dot(x_ref[...], w_ref[...],
                         preferred_element_type=jnp.float32) + b_ref[...]


def _h0(x, W1, b1r):
    return pl.pallas_call(
        _h0_body,
        grid=(NB_N,),
        in_specs=[
            pl.BlockSpec((BN, DIM_IN), lambda i: (i, 0)),
            pl.BlockSpec((DIM_IN, DN), lambda i: (0, 0)),
            pl.BlockSpec((1, DN), lambda i: (0, 0)),
        ],
        out_specs=pl.BlockSpec((BN, DN), lambda i: (i, 0)),
        out_shape=jax.ShapeDtypeStruct((N, DN), jnp.float32),
    )(x, W1, b1r)


def _mlp_msg_body(ea_ref, hs_ref, wk1, bk1, wk2, bk2, wk3, bk3, S, R,
                  w_out, msg_out):
    a1 = jnp.maximum(
        jnp.dot(ea_ref[...], wk1[...], preferred_element_type=jnp.float32)
        + bk1[...], 0.0)
    a2 = jnp.maximum(
        jnp.dot(a1, wk2[...], preferred_element_type=jnp.float32)
        + bk2[...], 0.0)
    w = jnp.dot(a2, wk3[...], preferred_element_type=jnp.float32) + bk3[...]
    w_out[...] = w
    hsbig = jnp.dot(hs_ref[...], S[...], preferred_element_type=jnp.float32)
    msg_out[...] = jnp.dot(hsbig * w, R[...],
                           preferred_element_type=jnp.float32)


def _mlp_msg(ea, hs, Wk1, bk1r, Wk2, bk2r, Wk3, bk3r, S, R):
    full = lambda a: pl.BlockSpec(a.shape, lambda i: tuple(0 for _ in a.shape))
    return pl.pallas_call(
        _mlp_msg_body,
        grid=(NBE,),
        in_specs=[
            pl.BlockSpec((BE, DN), lambda i: (i, 0)),
            pl.BlockSpec((BE, DN), lambda i: (i, 0)),
            full(Wk1), full(bk1r), full(Wk2), full(bk2r), full(Wk3),
            full(bk3r), full(S), full(R),
        ],
        out_specs=[
            pl.BlockSpec((BE, DN * DN), lambda i: (i, 0)),
            pl.BlockSpec((BE, DN), lambda i: (i, 0)),
        ],
        out_shape=[
            jax.ShapeDtypeStruct((E, DN * DN), jnp.float32),
            jax.ShapeDtypeStruct((E, DN), jnp.float32),
        ],
    )(ea, hs, Wk1, bk1r, Wk2, bk2r, Wk3, bk3r, S, R)


def _msg_body(w_ref, hs_ref, S, R, msg_out):
    hsbig = jnp.dot(hs_ref[...], S[...], preferred_element_type=jnp.float32)
    msg_out[...] = jnp.dot(hsbig * w_ref[...], R[...],
                           preferred_element_type=jnp.float32)


def _msg(w, hs, S, R):
    full = lambda a: pl.BlockSpec(a.shape, lambda i: tuple(0 for _ in a.shape))
    return pl.pallas_call(
        _msg_body,
        grid=(NBE,),
        in_specs=[
            pl.BlockSpec((BE, DN * DN), lambda i: (i, 0)),
            pl.BlockSpec((BE, DN), lambda i: (i, 0)),
            full(S), full(R),
        ],
        out_specs=pl.BlockSpec((BE, DN), lambda i: (i, 0)),
        out_shape=jax.ShapeDtypeStruct((E, DN), jnp.float32),
    )(w, hs, S, R)


def _update_body(s0, s1, c0, c1, h_ref, root, cb, o_ref):
    cnt = jnp.maximum(c0[...] + c1[...], 1.0)
    aggr = (s0[...] + s1[...]) / cnt
    hr = jnp.dot(h_ref[...], root[...], preferred_element_type=jnp.float32)
    o_ref[...] = jnp.maximum(aggr + hr + cb[...], 0.0)


def _update(s0, s1, c0, c1, h, root, cbr):
    blk = lambda: pl.BlockSpec((BN, DN), lambda i: (i, 0))
    full = lambda a: pl.BlockSpec(a.shape, lambda i: tuple(0 for _ in a.shape))
    return pl.pallas_call(
        _update_body,
        grid=(NB_N,),
        in_specs=[blk(), blk(), blk(), blk(), blk(), full(root), full(cbr)],
        out_specs=blk(),
        out_shape=jax.ShapeDtypeStruct((N, DN), jnp.float32),
    )(s0, s1, c0, c1, h, root, cbr)


def _pool_body(h_ref, b_ref, w2, b2, o_ref):
    ids = lax.broadcasted_iota(jnp.int32, (G, NP), 0)
    oh = (ids == b_ref[...]).astype(jnp.float32)
    s = jnp.dot(oh, h_ref[...], preferred_element_type=jnp.float32)
    cnt = jnp.maximum(jnp.sum(oh, axis=1, keepdims=True), 1.0)
    o_ref[...] = jnp.dot(s / cnt, w2[...],
                         preferred_element_type=jnp.float32) + b2[...]


def _pool(hp, bp, W2, b2r):
    full = lambda a: pl.BlockSpec(a.shape, lambda: tuple(0 for _ in a.shape))
    return pl.pallas_call(
        _pool_body,
        in_specs=[full(hp), full(bp), full(W2), full(b2r)],
        out_specs=pl.BlockSpec((G, 1), lambda: (0, 0)),
        out_shape=jax.ShapeDtypeStruct((G, 1), jnp.float32),
    )(hp, bp, W2, b2r)


_MESH = plsc.VectorSubcoreMesh(core_axis_name="c", subcore_axis_name="s")
_SC_PARAMS = pltpu.CompilerParams(use_tc_tiling_on_sc=False)


@functools.partial(
    pl.kernel,
    out_type=jax.ShapeDtypeStruct((E, DN), jnp.float32),
    mesh=_MESH,
    compiler_params=_SC_PARAMS,
    scratch_types=[
        pltpu.VMEM((CH,), jnp.int32),
        pltpu.VMEM((CH, DN), jnp.float32),
        pltpu.SemaphoreType.DMA,
    ],
)
def _gather_k(h_hbm, src_hbm, out_hbm, idx_v, rows_v, sem):
    cid = lax.axis_index("c")
    sid = lax.axis_index("s")
    wid = sid * 2 + cid
    for j in range(NCH):
        base = wid * EPW + j * CH
        pltpu.sync_copy(src_hbm.at[pl.ds(base, CH)], idx_v)
        pltpu.async_copy(h_hbm.at[idx_v], rows_v, sem).wait()
        pltpu.sync_copy(rows_v, out_hbm.at[pl.ds(base, CH)])


def _make_scatter(with_cnt):
    outs = (jax.ShapeDtypeStruct((2, N, DN), jnp.float32),)
    scratch = [
        pltpu.VMEM((CH,), jnp.int32),
        pltpu.VMEM((CH, DN), jnp.float32),
        pltpu.VMEM_SHARED((N, DN), jnp.float32),
    ]
    if with_cnt:
        outs = outs + (jax.ShapeDtypeStruct((2, N, DN), jnp.float32),)
        scratch += [
            pltpu.VMEM((CH, DN), jnp.float32),
            pltpu.VMEM_SHARED((N, DN), jnp.float32),
        ]

    @functools.partial(pl.kernel, out_type=outs, mesh=_MESH,
                       compiler_params=_SC_PARAMS, scratch_types=scratch)
    def _scatter_k(msg_hbm, dst_hbm, zeros_hbm, ones_hbm, *rest):
        if with_cnt:
            s_out, c_out, idx_v, rows_v, s_sh, ones_v, c_sh = rest
        else:
            s_out, idx_v, rows_v, s_sh = rest
        cid = lax.axis_index("c")
        sid = lax.axis_index("s")
        wid = sid * 2 + cid

        @pl.when(sid == 0)
        def _():
            pltpu.sync_copy(zeros_hbm, s_sh)
            if with_cnt:
                pltpu.sync_copy(zeros_hbm, c_sh)

        if with_cnt:
            pltpu.sync_copy(ones_hbm, ones_v)
        plsc.subcore_barrier()
        for j in range(NCH):
            base = wid * EPW + j * CH
            pltpu.sync_copy(dst_hbm.at[pl.ds(base, CH)], idx_v)
            pltpu.sync_copy(msg_hbm.at[pl.ds(base, CH)], rows_v)
            pltpu.sync_copy(rows_v, s_sh.at[idx_v], add=True)
            if with_cnt:
                pltpu.sync_copy(ones_v, c_sh.at[idx_v], add=True)
        plsc.subcore_barrier()

        @pl.when(sid == 0)
        def _():
            pltpu.sync_copy(s_sh, s_out.at[cid])
            if with_cnt:
                pltpu.sync_copy(c_sh, c_out.at[cid])

    return _scatter_k


_scatter_cnt_k = _make_scatter(True)
_scatter_k = _make_scatter(False)


def kernel(x, edge_index, edge_attr, batch, W1, b1, Wk1, bk1, Wk2, bk2,
           Wk3, bk3, root, cbias, W2, b2):
    src = edge_index[0]
    dst = edge_index[1]

    b1r = b1.reshape(1, DN)
    bk1r = bk1.reshape(1, 64)
    bk2r = bk2.reshape(1, 96)
    bk3r = bk3.reshape(1, DN * DN)
    cbr = cbias.reshape(1, DN)
    b2r = b2.reshape(1, 1)

    j = jnp.arange(DN * DN)
    S = (j[None, :] // DN == jnp.arange(DN)[:, None]).astype(jnp.float32)
    R = (j[:, None] % DN == jnp.arange(DN)[None, :]).astype(jnp.float32)

    zeros = jnp.zeros((N, DN), jnp.float32)
    ones = jnp.ones((CH, DN), jnp.float32)

    h0 = _h0(x, W1, b1r)
    hs0 = _gather_k(h0, src)
    w, msg1 = _mlp_msg(edge_attr, hs0, Wk1, bk1r, Wk2, bk2r, Wk3, bk3r, S, R)
    s1, c1 = _scatter_cnt_k(msg1, dst, zeros, ones)
    h1 = _update(s1[0], s1[1], c1[0], c1[1], h0, root, cbr)
    hs1 = _gather_k(h1, src)
    msg2 = _msg(w, hs1, S, R)
    (s2,) = _scatter_k(msg2, dst, zeros, ones)
    h2 = _update(s2[0], s2[1], c1[0], c1[1], h1, root, cbr)

    hp = jnp.pad(h2, ((0, NP - N), (0, 0)))
    bp = jnp.pad(batch, (0, NP - N), constant_values=-1).reshape(1, NP)
    return _pool(hp, bp, W2, b2r)


# speedup vs baseline: 5.6325x; 1.5924x over previous
import functools

import jax
import jax.numpy as jnp
from jax import lax
from jax.experimental import pallas as pl
from jax.experimental.pallas import tpu as pltpu
from jax.experimental.pallas import tpu_sc as plsc

N = 10000
E = 320000
G = 16
DIM_IN = 128
DN = 16

NW = 32
EPW = E // NW
CH = 2000
NCH = EPW // CH

NB_N = 10
BN = N // NB_N
BE = 3200
NBE = E // BE

NP = 10112


def _f32(x):
    return jnp.dot(x[0], x[1], preferred_element_type=jnp.float32)


def _h0_body(x_ref, w_ref, b_ref, o_ref):
    o_ref[...] = jnp.dot(x_ref[...], w_ref[...],
                         preferred_element_type=jnp.float32) + b_ref[...]


def _h0(x, W1, b1r):
    return pl.pallas_call(
        _h0_body,
        grid=(NB_N,),
        in_specs=[
            pl.BlockSpec((BN, DIM_IN), lambda i: (i, 0)),
            pl.BlockSpec((DIM_IN, DN), lambda i: (0, 0)),
            pl.BlockSpec((1, DN), lambda i: (0, 0)),
        ],
        out_specs=pl.BlockSpec((BN, DN), lambda i: (i, 0)),
        out_shape=jax.ShapeDtypeStruct((N, DN), jnp.float32),
    )(x, W1, b1r)


def _mlp_msg_body(ea_ref, hs_ref, wk1, bk1, wk2, bk2, wk3, bk3, S, R,
                  w_out, msg_out):
    bf = jnp.bfloat16
    a1 = jnp.maximum(
        jnp.dot(ea_ref[...].astype(bf), wk1[...],
                preferred_element_type=jnp.float32) + bk1[...], 0.0)
    a2 = jnp.maximum(
        jnp.dot(a1.astype(bf), wk2[...],
                preferred_element_type=jnp.float32) + bk2[...], 0.0)
    w = (jnp.dot(a2.astype(bf), wk3[...],
                 preferred_element_type=jnp.float32)
         + bk3[...]).astype(bf)
    w_out[...] = w
    hsbig = jnp.dot(hs_ref[...].astype(bf), S[...],
                    preferred_element_type=jnp.float32)
    msg_out[...] = jnp.dot((hsbig * w).astype(bf), R[...],
                           preferred_element_type=jnp.float32)


def _mlp_msg(ea, hs, Wk1, bk1r, Wk2, bk2r, Wk3, bk3r, S, R):
    full = lambda a: pl.BlockSpec(a.shape, lambda i: tuple(0 for _ in a.shape))
    return pl.pallas_call(
        _mlp_msg_body,
        grid=(NBE,),
        in_specs=[
            pl.BlockSpec((BE // 8, 128), lambda i: (i, 0)),
            pl.BlockSpec((BE // 8, 128), lambda i: (i, 0)),
            full(Wk1), full(bk1r), full(Wk2), full(bk2r), full(Wk3),
            full(bk3r), full(S), full(R),
        ],
        out_specs=[
            pl.BlockSpec((BE // 8, 8 * DN * DN), lambda i: (i, 0)),
            pl.BlockSpec((BE // 8, 128), lambda i: (i, 0)),
        ],
        out_shape=[
            jax.ShapeDtypeStruct((E // 8, 8 * DN * DN), jnp.bfloat16),
            jax.ShapeDtypeStruct((E // 8, 128), jnp.float32),
        ],
    )(ea, hs, Wk1, bk1r, Wk2, bk2r, Wk3, bk3r, S, R)


def _msg_body(w_ref, hs_ref, S, R, msg_out):
    bf = jnp.bfloat16
    hsbig = jnp.dot(hs_ref[...].astype(bf), S[...],
                    preferred_element_type=jnp.float32)
    msg_out[...] = jnp.dot((hsbig * w_ref[...]).astype(bf), R[...],
                           preferred_element_type=jnp.float32)


def _msg(w, hs, S, R):
    full = lambda a: pl.BlockSpec(a.shape, lambda i: tuple(0 for _ in a.shape))
    return pl.pallas_call(
        _msg_body,
        grid=(NBE,),
        in_specs=[
            pl.BlockSpec((BE // 8, 8 * DN * DN), lambda i: (i, 0)),
            pl.BlockSpec((BE // 8, 128), lambda i: (i, 0)),
            full(S), full(R),
        ],
        out_specs=pl.BlockSpec((BE // 8, 128), lambda i: (i, 0)),
        out_shape=jax.ShapeDtypeStruct((E // 8, 128), jnp.float32),
    )(w, hs, S, R)


def _update_body(s0, s1, c0, c1, h_ref, root, cb, o_ref):
    cnt = jnp.maximum(c0[...] + c1[...], 1.0)
    aggr = (s0[...] + s1[...]) / cnt
    hr = jnp.dot(h_ref[...], root[...], preferred_element_type=jnp.float32)
    o_ref[...] = jnp.maximum(aggr + hr + cb[...], 0.0)


def _update(s0, s1, c0, c1, h, root, cbr):
    blk = lambda: pl.BlockSpec((BN, DN), lambda i: (i, 0))
    full = lambda a: pl.BlockSpec(a.shape, lambda i: tuple(0 for _ in a.shape))
    return pl.pallas_call(
        _update_body,
        grid=(NB_N,),
        in_specs=[blk(), blk(), blk(), blk(), blk(), full(root), full(cbr)],
        out_specs=blk(),
        out_shape=jax.ShapeDtypeStruct((N, DN), jnp.float32),
    )(s0, s1, c0, c1, h, root, cbr)


def _pool_body(h_ref, b_ref, w2, b2, o_ref):
    ids = lax.broadcasted_iota(jnp.int32, (G, NP), 0)
    oh = (ids == b_ref[...]).astype(jnp.float32)
    s = jnp.dot(oh, h_ref[...], preferred_element_type=jnp.float32)
    cnt = jnp.maximum(jnp.sum(oh, axis=1, keepdims=True), 1.0)
    o_ref[...] = jnp.dot(s / cnt, w2[...],
                         preferred_element_type=jnp.float32) + b2[...]


def _pool(hp, bp, W2, b2r):
    full = lambda a: pl.BlockSpec(a.shape, lambda: tuple(0 for _ in a.shape))
    return pl.pallas_call(
        _pool_body,
        in_specs=[full(hp), full(bp), full(W2), full(b2r)],
        out_specs=pl.BlockSpec((G, 1), lambda: (0, 0)),
        out_shape=jax.ShapeDtypeStruct((G, 1), jnp.float32),
    )(hp, bp, W2, b2r)


_MESH = plsc.VectorSubcoreMesh(core_axis_name="c", subcore_axis_name="s")
_SC_PARAMS = pltpu.CompilerParams(use_tc_tiling_on_sc=False)


@functools.partial(
    pl.kernel,
    out_type=jax.ShapeDtypeStruct((E, DN), jnp.float32),
    mesh=_MESH,
    compiler_params=_SC_PARAMS,
    scratch_types=[
        pltpu.VMEM((CH,), jnp.int32),
        pltpu.VMEM((CH, DN), jnp.float32),
        pltpu.SemaphoreType.DMA,
    ],
)
def _gather_k(h_hbm, src_hbm, out_hbm, idx_v, rows_v, sem):
    cid = lax.axis_index("c")
    sid = lax.axis_index("s")
    wid = sid * 2 + cid
    for j in range(NCH):
        base = wid * EPW + j * CH
        pltpu.sync_copy(src_hbm.at[pl.ds(base, CH)], idx_v)
        pltpu.async_copy(h_hbm.at[idx_v], rows_v, sem).wait()
        pltpu.sync_copy(rows_v, out_hbm.at[pl.ds(base, CH)])


def _make_scatter(with_cnt):
    outs = (jax.ShapeDtypeStruct((2, N, DN), jnp.float32),)
    scratch = [
        pltpu.VMEM((CH,), jnp.int32),
        pltpu.VMEM((CH, DN), jnp.float32),
        pltpu.VMEM_SHARED((N, DN), jnp.float32),
    ]
    if with_cnt:
        outs = outs + (jax.ShapeDtypeStruct((2, N, DN), jnp.float32),)
        scratch += [
            pltpu.VMEM((CH, DN), jnp.float32),
            pltpu.VMEM_SHARED((N, DN), jnp.float32),
        ]

    @functools.partial(pl.kernel, out_type=outs, mesh=_MESH,
                       compiler_params=_SC_PARAMS, scratch_types=scratch)
    def _scatter_k(msg_hbm, dst_hbm, zeros_hbm, ones_hbm, *rest):
        if with_cnt:
            s_out, c_out, idx_v, rows_v, s_sh, ones_v, c_sh = rest
        else:
            s_out, idx_v, rows_v, s_sh = rest
        cid = lax.axis_index("c")
        sid = lax.axis_index("s")
        wid = sid * 2 + cid

        @pl.when(sid == 0)
        def _():
            pltpu.sync_copy(zeros_hbm, s_sh)
            if with_cnt:
                pltpu.sync_copy(zeros_hbm, c_sh)

        if with_cnt:
            pltpu.sync_copy(ones_hbm, ones_v)
        plsc.subcore_barrier()
        for j in range(NCH):
            base = wid * EPW + j * CH
            pltpu.sync_copy(dst_hbm.at[pl.ds(base, CH)], idx_v)
            pltpu.sync_copy(msg_hbm.at[pl.ds(base, CH)], rows_v)
            pltpu.sync_copy(rows_v, s_sh.at[idx_v], add=True)
            if with_cnt:
                pltpu.sync_copy(ones_v, c_sh.at[idx_v], add=True)
        plsc.subcore_barrier()

        @pl.when(sid == 0)
        def _():
            pltpu.sync_copy(s_sh, s_out.at[cid])
            if with_cnt:
                pltpu.sync_copy(c_sh, c_out.at[cid])

    return _scatter_k


_scatter_cnt_k = _make_scatter(True)
_scatter_k = _make_scatter(False)


def kernel(x, edge_index, edge_attr, batch, W1, b1, Wk1, bk1, Wk2, bk2,
           Wk3, bk3, root, cbias, W2, b2):
    src = edge_index[0]
    dst = edge_index[1]

    b1r = b1.reshape(1, DN)
    cbr = cbias.reshape(1, DN)
    b2r = b2.reshape(1, 1)

    j = jnp.arange(DN * DN)
    S0 = (j[None, :] // DN == jnp.arange(DN)[:, None]).astype(jnp.float32)
    R0 = (j[:, None] % DN == jnp.arange(DN)[None, :]).astype(jnp.float32)
    I8 = jnp.eye(8, dtype=jnp.float32)
    kr = lambda W: jnp.kron(I8, W).astype(jnp.bfloat16)
    S = kr(S0)
    R = kr(R0)
    Wk1b = kr(Wk1)
    Wk2b = kr(Wk2)
    Wk3b = kr(Wk3)
    bk1r = jnp.tile(bk1, 8).reshape(1, 8 * 64)
    bk2r = jnp.tile(bk2, 8).reshape(1, 8 * 96)
    bk3r = jnp.tile(bk3, 8).reshape(1, 8 * DN * DN)

    zeros = jnp.zeros((N, DN), jnp.float32)
    ones = jnp.ones((CH, DN), jnp.float32)

    eaP = edge_attr.reshape(E // 8, 128)
    h0 = _h0(x, W1, b1r)
    hs0 = _gather_k(h0, src).reshape(E // 8, 128)
    w, msg1 = _mlp_msg(eaP, hs0, Wk1b, bk1r, Wk2b, bk2r, Wk3b, bk3r,
                       S, R)
    s1, c1 = _scatter_cnt_k(msg1.reshape(E, DN), dst, zeros, ones)
    h1 = _update(s1[0], s1[1], c1[0], c1[1], h0, root, cbr)
    hs1 = _gather_k(h1, src).reshape(E // 8, 128)
    msg2 = _msg(w, hs1, S, R)
    (s2,) = _scatter_k(msg2.reshape(E, DN), dst, zeros, ones)
    h2 = _update(s2[0], s2[1], c1[0], c1[1], h1, root, cbr)

    hp = jnp.pad(h2, ((0, NP - N), (0, 0)))
    bp = jnp.pad(batch, (0, NP - N), constant_values=-1).reshape(1, NP)
    return _pool(hp, bp, W2, b2r)


# speedup vs baseline: 6.3884x; 1.1342x over previous
import functools

import jax
import jax.numpy as jnp
from jax import lax
from jax.experimental import pallas as pl
from jax.experimental.pallas import tpu as pltpu
from jax.experimental.pallas import tpu_sc as plsc

N = 10000
E = 320000
G = 16
DIM_IN = 128
DN = 16

NW = 32
EPW = E // NW
CH = 2000
NCH = EPW // CH

NB_N = 10
BN = N // NB_N
BE = 3200
NBE = E // BE

NP = 10112


def _f32(x):
    return jnp.dot(x[0], x[1], preferred_element_type=jnp.float32)


def _h0_body(x_ref, w_ref, b_ref, o_ref):
    o_ref[...] = jnp.dot(x_ref[...], w_ref[...],
                         preferred_element_type=jnp.float32) + b_ref[...]


def _h0(xp, W1B, b1B):
    full = lambda a: pl.BlockSpec(a.shape, lambda: tuple(0 for _ in a.shape))
    return pl.pallas_call(
        _h0_body,
        in_specs=[full(xp), full(W1B), full(b1B)],
        out_specs=pl.BlockSpec((N // 8, 128), lambda: (0, 0)),
        out_shape=jax.ShapeDtypeStruct((N // 8, 128), jnp.float32),
    )(xp, W1B, b1B)


def _l3_w(a2b, wk3, bk3):
    parts = [
        jnp.dot(a2b[:, e * 96:(e + 1) * 96], wk3[...],
                preferred_element_type=jnp.float32)
        for e in range(8)
    ]
    return (jnp.concatenate(parts, axis=1) + bk3[...]).astype(jnp.bfloat16)


def _mlp_msg_body(ea_ref, hs_ref, wk1, bk1, wk2, bk2, wk3, bk3, S, R,
                  a2_out, msg_out):
    bf = jnp.bfloat16
    a1 = jnp.maximum(
        jnp.dot(ea_ref[...].astype(bf), wk1[...],
                preferred_element_type=jnp.float32) + bk1[...], 0.0)
    a2 = jnp.maximum(
        jnp.dot(a1.astype(bf), wk2[...],
                preferred_element_type=jnp.float32) + bk2[...], 0.0).astype(bf)
    a2_out[...] = a2
    w = _l3_w(a2, wk3, bk3)
    hsbig = jnp.dot(hs_ref[...].astype(bf), S[...],
                    preferred_element_type=jnp.float32)
    msg_out[...] = jnp.dot((hsbig * w).astype(bf), R[...],
                           preferred_element_type=jnp.float32)


def _mlp_msg(ea, hs, Wk1, bk1r, Wk2, bk2r, Wk3, bk3r, S, R):
    full = lambda a: pl.BlockSpec(a.shape, lambda i: tuple(0 for _ in a.shape))
    return pl.pallas_call(
        _mlp_msg_body,
        grid=(NBE,),
        in_specs=[
            pl.BlockSpec((BE // 8, 128), lambda i: (i, 0)),
            pl.BlockSpec((BE // 8, 128), lambda i: (i, 0)),
            full(Wk1), full(bk1r), full(Wk2), full(bk2r), full(Wk3),
            full(bk3r), full(S), full(R),
        ],
        out_specs=[
            pl.BlockSpec((BE // 8, 8 * 96), lambda i: (i, 0)),
            pl.BlockSpec((BE // 8, 128), lambda i: (i, 0)),
        ],
        out_shape=[
            jax.ShapeDtypeStruct((E // 8, 8 * 96), jnp.bfloat16),
            jax.ShapeDtypeStruct((E // 8, 128), jnp.float32),
        ],
    )(ea, hs, Wk1, bk1r, Wk2, bk2r, Wk3, bk3r, S, R)


def _msg_body(a2_ref, hs_ref, wk3, bk3, S, R, msg_out):
    bf = jnp.bfloat16
    w = _l3_w(a2_ref[...], wk3, bk3)
    hsbig = jnp.dot(hs_ref[...].astype(bf), S[...],
                    preferred_element_type=jnp.float32)
    msg_out[...] = jnp.dot((hsbig * w).astype(bf), R[...],
                           preferred_element_type=jnp.float32)


def _msg(a2, hs, Wk3b, bk3r, S, R):
    full = lambda a: pl.BlockSpec(a.shape, lambda i: tuple(0 for _ in a.shape))
    return pl.pallas_call(
        _msg_body,
        grid=(NBE,),
        in_specs=[
            pl.BlockSpec((BE // 8, 8 * 96), lambda i: (i, 0)),
            pl.BlockSpec((BE // 8, 128), lambda i: (i, 0)),
            full(Wk3b), full(bk3r), full(S), full(R),
        ],
        out_specs=pl.BlockSpec((BE // 8, 128), lambda i: (i, 0)),
        out_shape=jax.ShapeDtypeStruct((E // 8, 128), jnp.float32),
    )(a2, hs, Wk3b, bk3r, S, R)


def _update_body(s_ref, c_ref, h_ref, root, cb, o_ref):
    cnt = jnp.maximum(c_ref[0] + c_ref[1], 1.0)
    aggr = (s_ref[0] + s_ref[1]) / cnt
    hr = jnp.dot(h_ref[...], root[...], preferred_element_type=jnp.float32)
    o_ref[...] = jnp.maximum(aggr + hr + cb[...], 0.0)


def _update(sp, cp, hp, rootB, cbB):
    full = lambda a: pl.BlockSpec(a.shape, lambda: tuple(0 for _ in a.shape))
    return pl.pallas_call(
        _update_body,
        in_specs=[full(sp), full(cp), full(hp), full(rootB), full(cbB)],
        out_specs=pl.BlockSpec((N // 8, 128), lambda: (0, 0)),
        out_shape=jax.ShapeDtypeStruct((N // 8, 128), jnp.float32),
    )(sp, cp, hp, rootB, cbB)


def _pool_body(h_ref, b_ref, w2, b2, o_ref):
    ids = lax.broadcasted_iota(jnp.int32, (G, NP), 0)
    oh = (ids == b_ref[...]).astype(jnp.float32)
    s = jnp.dot(oh, h_ref[...], preferred_element_type=jnp.float32)
    cnt = jnp.maximum(jnp.sum(oh, axis=1, keepdims=True), 1.0)
    o_ref[...] = jnp.dot(s / cnt, w2[...],
                         preferred_element_type=jnp.float32) + b2[...]


def _pool(hp, bp, W2, b2r):
    full = lambda a: pl.BlockSpec(a.shape, lambda: tuple(0 for _ in a.shape))
    return pl.pallas_call(
        _pool_body,
        in_specs=[full(hp), full(bp), full(W2), full(b2r)],
        out_specs=pl.BlockSpec((G, 1), lambda: (0, 0)),
        out_shape=jax.ShapeDtypeStruct((G, 1), jnp.float32),
    )(hp, bp, W2, b2r)


_MESH = plsc.VectorSubcoreMesh(core_axis_name="c", subcore_axis_name="s")
_SC_PARAMS = pltpu.CompilerParams(use_tc_tiling_on_sc=False)


@functools.partial(
    pl.kernel,
    out_type=jax.ShapeDtypeStruct((E, DN), jnp.float32),
    mesh=_MESH,
    compiler_params=_SC_PARAMS,
    scratch_types=[
        pltpu.VMEM((CH,), jnp.int32),
        pltpu.VMEM((CH, DN), jnp.float32),
        pltpu.SemaphoreType.DMA,
    ],
)
def _gather_k(h_hbm, src_hbm, out_hbm, idx_v, rows_v, sem):
    cid = lax.axis_index("c")
    sid = lax.axis_index("s")
    wid = sid * 2 + cid
    for j in range(NCH):
        base = wid * EPW + j * CH
        pltpu.sync_copy(src_hbm.at[pl.ds(base, CH)], idx_v)
        pltpu.async_copy(h_hbm.at[idx_v], rows_v, sem).wait()
        pltpu.sync_copy(rows_v, out_hbm.at[pl.ds(base, CH)])


def _make_scatter(with_cnt):
    outs = (jax.ShapeDtypeStruct((2, N, DN), jnp.float32),)
    scratch = [
        pltpu.VMEM((CH,), jnp.int32),
        pltpu.VMEM((CH, DN), jnp.float32),
        pltpu.VMEM_SHARED((N, DN), jnp.float32),
    ]
    if with_cnt:
        outs = outs + (jax.ShapeDtypeStruct((2, N, DN), jnp.float32),)
        scratch += [
            pltpu.VMEM((CH, DN), jnp.float32),
            pltpu.VMEM_SHARED((N, DN), jnp.float32),
        ]

    @functools.partial(pl.kernel, out_type=outs, mesh=_MESH,
                       compiler_params=_SC_PARAMS, scratch_types=scratch)
    def _scatter_k(msg_hbm, dst_hbm, zeros_hbm, ones_hbm, *rest):
        if with_cnt:
            s_out, c_out, idx_v, rows_v, s_sh, ones_v, c_sh = rest
        else:
            s_out, idx_v, rows_v, s_sh = rest
        cid = lax.axis_index("c")
        sid = lax.axis_index("s")
        wid = sid * 2 + cid

        @pl.when(sid == 0)
        def _():
            pltpu.sync_copy(zeros_hbm, s_sh)
            if with_cnt:
                pltpu.sync_copy(zeros_hbm, c_sh)

        if with_cnt:
            pltpu.sync_copy(ones_hbm, ones_v)
        plsc.subcore_barrier()
        for j in range(NCH):
            base = wid * EPW + j * CH
            pltpu.sync_copy(dst_hbm.at[pl.ds(base, CH)], idx_v)
            pltpu.sync_copy(msg_hbm.at[pl.ds(base, CH)], rows_v)
            pltpu.sync_copy(rows_v, s_sh.at[idx_v], add=True)
            if with_cnt:
                pltpu.sync_copy(ones_v, c_sh.at[idx_v], add=True)
        plsc.subcore_barrier()

        @pl.when(sid == 0)
        def _():
            pltpu.sync_copy(s_sh, s_out.at[cid])
            if with_cnt:
                pltpu.sync_copy(c_sh, c_out.at[cid])

    return _scatter_k


_scatter_cnt_k = _make_scatter(True)
_scatter_k = _make_scatter(False)


def kernel(x, edge_index, edge_attr, batch, W1, b1, Wk1, bk1, Wk2, bk2,
           Wk3, bk3, root, cbias, W2, b2):
    src = edge_index[0]
    dst = edge_index[1]

    b2r = b2.reshape(1, 1)

    j = jnp.arange(DN * DN)
    S0 = (j[None, :] // DN == jnp.arange(DN)[:, None]).astype(jnp.float32)
    R0 = (j[:, None] % DN == jnp.arange(DN)[None, :]).astype(jnp.float32)
    I8 = jnp.eye(8, dtype=jnp.float32)
    kr = lambda W: jnp.kron(I8, W).astype(jnp.bfloat16)
    S = kr(S0)
    R = kr(R0)
    Wk1b = kr(Wk1)
    Wk2b = kr(Wk2)
    Wk3b = Wk3.astype(jnp.bfloat16)
    bk1r = jnp.tile(bk1, 8).reshape(1, 8 * 64)
    bk2r = jnp.tile(bk2, 8).reshape(1, 8 * 96)
    bk3r = jnp.tile(bk3, 8).reshape(1, 8 * DN * DN)
    W1B = jnp.kron(I8, W1)
    b1B = jnp.tile(b1, 8).reshape(1, 128)
    rootB = jnp.kron(I8, root)
    cbB = jnp.tile(cbias, 8).reshape(1, 128)

    zeros = jnp.zeros((N // 8, 128), jnp.float32).reshape(N, DN)
    ones = jnp.ones((CH // 8, 128), jnp.float32).reshape(CH, DN)

    eaP = edge_attr.reshape(E // 8, 128)
    xp = x.reshape(N // 8, 8 * DIM_IN)
    h0 = _h0(xp, W1B, b1B)
    hs0 = _gather_k(h0.reshape(N, DN), src).reshape(E // 8, 128)
    a2, msg1 = _mlp_msg(eaP, hs0, Wk1b, bk1r, Wk2b, bk2r, Wk3b, bk3r,
                        S, R)
    s1, c1 = _scatter_cnt_k(msg1.reshape(E, DN), dst, zeros, ones)
    s1p = s1.reshape(2, N // 8, 128)
    c1p = c1.reshape(2, N // 8, 128)
    h1 = _update(s1p, c1p, h0, rootB, cbB)
    hs1 = _gather_k(h1.reshape(N, DN), src).reshape(E // 8, 128)
    msg2 = _msg(a2, hs1, Wk3b, bk3r, S, R)
    (s2,) = _scatter_k(msg2.reshape(E, DN), dst, zeros, ones)
    h2 = _update(s2.reshape(2, N // 8, 128), c1p, h1, rootB, cbB)

    hp = jnp.pad(h2.reshape(N, DN), ((0, NP - N), (0, 0)))
    bp = jnp.pad(batch, (0, NP - N), constant_values=-1).reshape(1, NP)
    return _pool(hp, bp, W2, b2r)
